# baseline (speedup 1.0000x reference)
import functools
import math

import jax
import jax.numpy as jnp
from jax import lax
from jax.experimental import pallas as pl
from jax.experimental.pallas import tpu as pltpu
from jax.experimental.pallas import tpu_sc as plsc

D_MODEL = 64
SCALE = math.sqrt(D_MODEL)

NC = 2
NS = 16
NW = NC * NS

CHUNK = 512


def _emb_body(b_per_w, table_hbm, idx_hbm, out_hbm, idx_v, rows_v, gsem):
    wid = lax.axis_index("s") * NC + lax.axis_index("c")
    base = wid * b_per_w
    pltpu.sync_copy(idx_hbm.at[pl.ds(base, b_per_w)], idx_v)

    n_chunks = b_per_w // CHUNK
    for g in range(n_chunks):
        pltpu.async_copy(
            table_hbm.at[idx_v.at[pl.ds(g * CHUNK, CHUNK)]], rows_v, gsem
        ).wait()

        def scale_row(i, _):
            for c in range(D_MODEL // 16):
                sl = pl.ds(c * 16, 16)
                rows_v[i, sl] = rows_v[i, sl] * SCALE
            return 0

        lax.fori_loop(0, CHUNK, scale_row, 0)

        pltpu.sync_copy(rows_v, out_hbm.at[pl.ds(base + g * CHUNK, CHUNK)])


def kernel(x, table):
    B = x.shape[0] * x.shape[1]
    b_per_w = B // NW
    assert b_per_w % CHUNK == 0

    mesh = plsc.VectorSubcoreMesh(
        core_axis_name="c", subcore_axis_name="s", num_cores=NC
    )
    k = pl.kernel(
        functools.partial(_emb_body, b_per_w),
        out_type=jax.ShapeDtypeStruct((B, D_MODEL), jnp.float32),
        mesh=mesh,
        compiler_params=pltpu.CompilerParams(use_tc_tiling_on_sc=False),
        scratch_types=[
            pltpu.VMEM((b_per_w,), jnp.int32),
            pltpu.VMEM((CHUNK, D_MODEL), jnp.float32),
            pltpu.SemaphoreType.DMA,
        ],
    )
    out = k(table, x.reshape(-1))
    return out.reshape(x.shape[0], x.shape[1], D_MODEL)


# speedup vs baseline: 1.1427x; 1.1427x over previous
import functools
import math

import jax
import jax.numpy as jnp
from jax import lax
from jax.experimental import pallas as pl
from jax.experimental.pallas import tpu as pltpu
from jax.experimental.pallas import tpu_sc as plsc

D_MODEL = 64
SCALE = math.sqrt(D_MODEL)
PADW = 128

NC = 2
NS = 16
NW = NC * NS

CHUNK = 512
NBUF = 3


def _gather_body(b_per_w, table_hbm, idx_hbm, raw_hbm, idx_v, bufs, gsem, wsem):
    wid = lax.axis_index("s") * NC + lax.axis_index("c")
    base = wid * b_per_w
    pltpu.sync_copy(idx_hbm.at[pl.ds(base, b_per_w)], idx_v)

    n = b_per_w // CHUNK

    def start_gather(g):
        b = g % NBUF
        return pltpu.async_copy(
            table_hbm.at[idx_v.at[pl.ds(g * CHUNK, CHUNK)]], bufs[b], gsem.at[b]
        )

    def start_write(g):
        b = g % NBUF
        dst = raw_hbm.at[pl.ds(base + g * CHUNK, CHUNK), pl.ds(0, D_MODEL)]
        return pltpu.async_copy(bufs[b], dst, wsem.at[b])

    ghandles = [None] * n
    whandles = [None] * n
    for g in range(n):
        if g >= NBUF:
            whandles[g - NBUF].wait()
        ghandles[g] = start_gather(g)
        if g >= 1:
            ghandles[g - 1].wait()
            whandles[g - 1] = start_write(g - 1)
    ghandles[n - 1].wait()
    whandles[n - 1] = start_write(n - 1)
    for g in range(n - NBUF + 1, n):
        whandles[g].wait()


def _scale_body(r_ref, o_ref):
    o_ref[...] = r_ref[:, :D_MODEL] * SCALE


def kernel(x, table):
    B = x.shape[0] * x.shape[1]
    b_per_w = B // NW
    assert b_per_w % CHUNK == 0

    mesh = plsc.VectorSubcoreMesh(
        core_axis_name="c", subcore_axis_name="s", num_cores=NC
    )
    gather = pl.kernel(
        functools.partial(_gather_body, b_per_w),
        out_type=jax.ShapeDtypeStruct((B, PADW), jnp.float32),
        mesh=mesh,
        compiler_params=pltpu.CompilerParams(use_tc_tiling_on_sc=False),
        scratch_types=[
            pltpu.VMEM((b_per_w,), jnp.int32),
            [pltpu.VMEM((CHUNK, D_MODEL), jnp.float32) for _ in range(NBUF)],
            pltpu.SemaphoreType.DMA((NBUF,)),
            pltpu.SemaphoreType.DMA((NBUF,)),
        ],
    )
    raw = gather(table, x.reshape(-1))

    ROWS = 4096
    out = pl.pallas_call(
        _scale_body,
        grid=(B // ROWS,),
        in_specs=[pl.BlockSpec((ROWS, PADW), lambda i: (i, 0))],
        out_specs=pl.BlockSpec((ROWS, D_MODEL), lambda i: (i, 0)),
        out_shape=jax.ShapeDtypeStruct((B, D_MODEL), jnp.float32),
    )(raw)
    return out.reshape(x.shape[0], x.shape[1], D_MODEL)
